# baseline (speedup 1.0000x reference)
import functools

import jax
import jax.numpy as jnp
from jax.experimental import pallas as pl
from jax.experimental.pallas import tpu as pltpu

_TILE = 5000
_CACHE_BLOCKS = 12


def _dot(a, b, dims):
    return jax.lax.dot_general(a, b, (dims, ((), ())),
                               preferred_element_type=jnp.float32)


def _leaky(v):
    return jnp.where(v >= 0, v, 0.01 * v)


def _fused_kernel(x_ref, e_hbm_ref,
                  we_ref, be_ref, wsi_ref, bsi_ref,
                  wso_ref, bso_ref, wo_ref, bo_ref,
                  skip_out_ref, to_gnn_ref,
                  ebuf_ref, esem, xcache_ref,
                  sec_ref, cs_ref, m_ref, l_ref, acc_ref,
                  mso_ref, mog_ref, vso_ref, vg_ref, *, g, tile, cb):
    i = pl.program_id(0)

    def _ecopy(blk, slot):
        pltpu.make_async_copy(
            e_hbm_ref.at[pl.ds(blk * tile, tile), :],
            ebuf_ref.at[slot],
            esem.at[slot],
        ).start()

    @pl.when(i < g)
    def _phase_a():
        @pl.when(i == 0)
        def _():
            _ecopy(0, 0)
            _ecopy(1, 1)

        slot = jax.lax.rem(i, 2)
        pltpu.make_async_copy(
            e_hbm_ref.at[pl.ds(i * tile, tile), :],
            ebuf_ref.at[slot],
            esem.at[slot],
        ).wait()

        @pl.when((i >= 1) & (i + 1 < g))
        def _():
            _ecopy(i + 1, jax.lax.rem(i + 1, 2))

        x = x_ref[...]

        @pl.when(i >= g - cb)
        def _():
            xcache_ref[pl.ds((i - (g - cb)) * tile, tile), :] = x

        e = ebuf_ref[slot]
        ones = jnp.ones((tile, 1), jnp.float32)
        part = _dot(e, x, ((0,), (0,)))
        cs_part = _dot(e, ones, ((0,), (0,)))

        @pl.when(i == 0)
        def _():
            sec_ref[...] = jnp.zeros_like(sec_ref)
            cs_ref[...] = jnp.zeros_like(cs_ref)

        sec_ref[...] += part
        cs_ref[...] += cs_part

        @pl.when(i == g - 1)
        def _():
            sec_ref[...] = sec_ref[...] / cs_ref[...]

    @pl.when((i >= g) & (i < 2 * g))
    def _phase_b():
        j = i - g
        sel = jnp.clip(j - (g - cb), 0, cb - 1)
        x = jnp.where(j < g - cb, x_ref[...],
                      xcache_ref[pl.ds(sel * tile, tile), :])
        s = _dot(sec_ref[...], x, ((1,), (1,)))
        tile_max = jnp.max(s, axis=1, keepdims=True)

        @pl.when(j == 0)
        def _():
            m_ref[...] = jnp.full_like(m_ref, -jnp.inf)
            l_ref[...] = jnp.zeros_like(l_ref)
            acc_ref[...] = jnp.zeros_like(acc_ref)

        m_old = m_ref[...]
        m_new = jnp.maximum(m_old, tile_max)
        corr = jnp.exp(m_old - m_new)
        p = jnp.exp(s - m_new)
        l_ref[...] = l_ref[...] * corr + jnp.sum(p, axis=1, keepdims=True)
        acc_ref[...] = acc_ref[...] * corr + _dot(p, x, ((1,), (0,)))
        m_ref[...] = m_new

        @pl.when(j == g - 1)
        def _():
            sec2 = acc_ref[...] / l_ref[...]
            acc_ref[...] = sec2
            t1 = _dot(sec2, we_ref[...], ((1,), (1,)))
            mso_ref[...] = _dot(t1, wso_ref[...], ((1,), (1,)))
            t2 = _dot(t1, wsi_ref[...], ((1,), (1,)))
            mog_ref[...] = _dot(t2, wo_ref[...], ((1,), (1,)))
            vso_ref[...] = _dot(be_ref[...], wso_ref[...], ((1,), (1,))) + bso_ref[...]
            b1 = _dot(be_ref[...], wsi_ref[...], ((1,), (1,))) + bsi_ref[...]
            vg_ref[...] = _dot(b1, wo_ref[...], ((1,), (1,))) + bo_ref[...]

    @pl.when(i >= 2 * g)
    def _phase_c():
        k = i - 2 * g
        sel = jnp.clip(k - (g - cb), 0, cb - 1)
        x = jnp.where(k < g - cb, x_ref[...],
                      xcache_ref[pl.ds(sel * tile, tile), :])
        logits = _dot(x, acc_ref[...], ((1,), (1,)))
        logits = logits - jnp.max(logits, axis=1, keepdims=True)
        p = jnp.exp(logits)
        inv = p / jnp.sum(p, axis=1, keepdims=True)
        skip_out_ref[...] = _leaky(_dot(inv, mso_ref[...], ((1,), (0,))) + vso_ref[...])
        to_gnn_ref[...] = _leaky(_dot(x, wo_ref[...], ((1,), (1,)))
                                 + _dot(inv, mog_ref[...], ((1,), (0,))) + vg_ref[...])


@jax.jit
def kernel(x, ent2sec_mat, W_ent, b_ent, W_skip_in, b_skip_in,
           W_skip_out, b_skip_out, W_out, b_out):
    n, d = x.shape
    s = ent2sec_mat.shape[1]
    tile = _TILE if n % _TILE == 0 else n
    g = n // tile

    cb = _CACHE_BLOCKS
    def x_tile(i):
        j = jnp.where(i < 2 * g, i - g, i - 2 * g)
        streamed = jnp.minimum(j, g - cb - 1)
        return (jnp.where(i < g, i, streamed), 0)
    out_tile = lambda i: (jnp.where(i >= 2 * g, i - 2 * g, 0), 0)
    whole = lambda i: (0, 0)

    bias2d = lambda b: b.reshape(1, d)
    wspec = pl.BlockSpec((d, d), whole)
    bspec = pl.BlockSpec((1, d), whole)
    sd = lambda: pltpu.VMEM((s, d), jnp.float32)
    s1 = lambda: pltpu.VMEM((s, 1), jnp.float32)
    v1 = lambda: pltpu.VMEM((1, d), jnp.float32)

    skip_out, to_gnn = pl.pallas_call(
        functools.partial(_fused_kernel, g=g, tile=tile, cb=cb),
        grid=(3 * g,),
        in_specs=[pl.BlockSpec((tile, d), x_tile),
                  pl.BlockSpec(memory_space=pl.ANY),
                  wspec, bspec, wspec, bspec, wspec, bspec, wspec, bspec],
        out_specs=[pl.BlockSpec((tile, d), out_tile),
                   pl.BlockSpec((tile, d), out_tile)],
        out_shape=[jax.ShapeDtypeStruct((n, d), jnp.float32),
                   jax.ShapeDtypeStruct((n, d), jnp.float32)],
        scratch_shapes=[pltpu.VMEM((2, tile, s), jnp.float32),
                        pltpu.SemaphoreType.DMA((2,)),
                        pltpu.VMEM((cb * tile, d), jnp.float32),
                        sd(), s1(), s1(), s1(), sd(), sd(), sd(), v1(), v1()],
    )(x, ent2sec_mat,
      W_ent, bias2d(b_ent), W_skip_in, bias2d(b_skip_in),
      W_skip_out, bias2d(b_skip_out), W_out, bias2d(b_out))

    return (skip_out, to_gnn)


# speedup vs baseline: 1.1767x; 1.1767x over previous
import functools

import jax
import jax.numpy as jnp
from jax.experimental import pallas as pl
from jax.experimental.pallas import tpu as pltpu

_TILE = 10000
_CACHE_BLOCKS = 2


def _dot(a, b, dims):
    return jax.lax.dot_general(a, b, (dims, ((), ())),
                               preferred_element_type=jnp.float32)


def _leaky(v):
    return jnp.where(v >= 0, v, 0.01 * v)


def _fused_kernel(x_ref, e_hbm_ref,
                  we_ref, be_ref, wsi_ref, bsi_ref,
                  wso_ref, bso_ref, wo_ref, bo_ref,
                  skip_out_ref, to_gnn_ref,
                  ebuf_ref, esem, xcache_ref,
                  sec_ref, cs_ref, m_ref, l_ref, acc_ref,
                  mso_ref, mog_ref, vso_ref, vg_ref, *, g, tile, cb):
    i = pl.program_id(0)

    def _ecopy(blk, slot):
        pltpu.make_async_copy(
            e_hbm_ref.at[pl.ds(blk * tile, tile), :],
            ebuf_ref.at[slot],
            esem.at[slot],
        ).start()

    @pl.when(i < g)
    def _phase_a():
        @pl.when(i == 0)
        def _():
            _ecopy(0, 0)
            _ecopy(1, 1)

        slot = jax.lax.rem(i, 2)
        pltpu.make_async_copy(
            e_hbm_ref.at[pl.ds(i * tile, tile), :],
            ebuf_ref.at[slot],
            esem.at[slot],
        ).wait()

        @pl.when((i >= 1) & (i + 1 < g))
        def _():
            _ecopy(i + 1, jax.lax.rem(i + 1, 2))

        x = x_ref[...]

        @pl.when(i >= g - cb)
        def _():
            xcache_ref[pl.ds((i - (g - cb)) * tile, tile), :] = x

        e = ebuf_ref[slot]
        ones = jnp.ones((tile, 1), jnp.float32)
        part = _dot(e, x, ((0,), (0,)))
        cs_part = _dot(e, ones, ((0,), (0,)))

        @pl.when(i == 0)
        def _():
            sec_ref[...] = jnp.zeros_like(sec_ref)
            cs_ref[...] = jnp.zeros_like(cs_ref)

        sec_ref[...] += part
        cs_ref[...] += cs_part

        @pl.when(i == g - 1)
        def _():
            sec_ref[...] = sec_ref[...] / cs_ref[...]

    @pl.when((i >= g) & (i < 2 * g))
    def _phase_b():
        j = i - g

        @pl.when(j == 0)
        def _():
            m_ref[...] = jnp.full_like(m_ref, -jnp.inf)
            l_ref[...] = jnp.zeros_like(l_ref)
            acc_ref[...] = jnp.zeros_like(acc_ref)

        def body(x):
            s = _dot(sec_ref[...], x, ((1,), (1,)))
            tile_max = jnp.max(s, axis=1, keepdims=True)
            m_old = m_ref[...]
            m_new = jnp.maximum(m_old, tile_max)
            corr = jnp.exp(m_old - m_new)
            p = jnp.exp(s - m_new)
            l_ref[...] = l_ref[...] * corr + jnp.sum(p, axis=1, keepdims=True)
            acc_ref[...] = acc_ref[...] * corr + _dot(p, x, ((1,), (0,)))
            m_ref[...] = m_new

        @pl.when(j < g - cb)
        def _():
            body(x_ref[...])

        @pl.when(j >= g - cb)
        def _():
            sel = jnp.clip(j - (g - cb), 0, cb - 1)
            body(xcache_ref[pl.ds(sel * tile, tile), :])

        @pl.when(j == g - 1)
        def _():
            sec2 = acc_ref[...] / l_ref[...]
            acc_ref[...] = sec2
            t1 = _dot(sec2, we_ref[...], ((1,), (1,)))
            mso_ref[...] = _dot(t1, wso_ref[...], ((1,), (1,)))
            t2 = _dot(t1, wsi_ref[...], ((1,), (1,)))
            mog_ref[...] = _dot(t2, wo_ref[...], ((1,), (1,)))
            vso_ref[...] = _dot(be_ref[...], wso_ref[...], ((1,), (1,))) + bso_ref[...]
            b1 = _dot(be_ref[...], wsi_ref[...], ((1,), (1,))) + bsi_ref[...]
            vg_ref[...] = _dot(b1, wo_ref[...], ((1,), (1,))) + bo_ref[...]

    @pl.when(i >= 2 * g)
    def _phase_c():
        k = i - 2 * g

        def body(x):
            logits = _dot(x, acc_ref[...], ((1,), (1,)))
            logits = logits - jnp.max(logits, axis=1, keepdims=True)
            p = jnp.exp(logits)
            inv = p / jnp.sum(p, axis=1, keepdims=True)
            skip_out_ref[...] = _leaky(_dot(inv, mso_ref[...], ((1,), (0,))) + vso_ref[...])
            to_gnn_ref[...] = _leaky(_dot(x, wo_ref[...], ((1,), (1,)))
                                     + _dot(inv, mog_ref[...], ((1,), (0,))) + vg_ref[...])

        @pl.when(k < g - cb)
        def _():
            body(x_ref[...])

        @pl.when(k >= g - cb)
        def _():
            sel = jnp.clip(k - (g - cb), 0, cb - 1)
            body(xcache_ref[pl.ds(sel * tile, tile), :])


@jax.jit
def kernel(x, ent2sec_mat, W_ent, b_ent, W_skip_in, b_skip_in,
           W_skip_out, b_skip_out, W_out, b_out):
    n, d = x.shape
    s = ent2sec_mat.shape[1]
    tile = _TILE if n % _TILE == 0 else n
    g = n // tile

    cb = _CACHE_BLOCKS
    def x_tile(i):
        j = jnp.where(i < 2 * g, i - g, i - 2 * g)
        streamed = jnp.minimum(j, g - cb - 1)
        return (jnp.where(i < g, i, streamed), 0)
    out_tile = lambda i: (jnp.where(i >= 2 * g, i - 2 * g, 0), 0)
    whole = lambda i: (0, 0)

    bias2d = lambda b: b.reshape(1, d)
    wspec = pl.BlockSpec((d, d), whole)
    bspec = pl.BlockSpec((1, d), whole)
    sd = lambda: pltpu.VMEM((s, d), jnp.float32)
    s1 = lambda: pltpu.VMEM((s, 1), jnp.float32)
    v1 = lambda: pltpu.VMEM((1, d), jnp.float32)

    skip_out, to_gnn = pl.pallas_call(
        functools.partial(_fused_kernel, g=g, tile=tile, cb=cb),
        grid=(3 * g,),
        in_specs=[pl.BlockSpec((tile, d), x_tile),
                  pl.BlockSpec(memory_space=pl.ANY),
                  wspec, bspec, wspec, bspec, wspec, bspec, wspec, bspec],
        out_specs=[pl.BlockSpec((tile, d), out_tile),
                   pl.BlockSpec((tile, d), out_tile)],
        out_shape=[jax.ShapeDtypeStruct((n, d), jnp.float32),
                   jax.ShapeDtypeStruct((n, d), jnp.float32)],
        scratch_shapes=[pltpu.VMEM((2, tile, s), jnp.float32),
                        pltpu.SemaphoreType.DMA((2,)),
                        pltpu.VMEM((cb * tile, d), jnp.float32),
                        sd(), s1(), s1(), s1(), sd(), sd(), sd(), v1(), v1()],
    )(x, ent2sec_mat,
      W_ent, bias2d(b_ent), W_skip_in, bias2d(b_skip_in),
      W_skip_out, bias2d(b_skip_out), W_out, bias2d(b_out))

    return (skip_out, to_gnn)


# speedup vs baseline: 1.1879x; 1.0095x over previous
import functools

import jax
import jax.numpy as jnp
from jax.experimental import pallas as pl
from jax.experimental.pallas import tpu as pltpu

_TILE = 10000


def _dot(a, b, dims):
    return jax.lax.dot_general(a, b, (dims, ((), ())),
                               preferred_element_type=jnp.float32)


def _leaky(v):
    return jnp.where(v >= 0, v, 0.01 * v)


def _fused_kernel(x_ref, e0_ref, e1_ref,
                  we_ref, be_ref, wsi_ref, bsi_ref,
                  wso_ref, bso_ref, wo_ref, bo_ref,
                  skip_out_ref, to_gnn_ref,
                  sec_ref, cs_ref, m_ref, l_ref, acc_ref,
                  mso_ref, mog_ref, vso_ref, vg_ref, *, g, tile):
    i = pl.program_id(0)

    @pl.when(i < g)
    def _phase_a():
        x = x_ref[...]
        q = tile // 2
        ones = jnp.ones((q, 1), jnp.float32)
        part = jnp.zeros_like(sec_ref)
        cs_part = jnp.zeros_like(cs_ref)
        for k, e_ref in enumerate((e0_ref, e1_ref)):
            e = e_ref[...]
            part += _dot(e, x[k * q:(k + 1) * q, :], ((0,), (0,)))
            cs_part += _dot(e, ones, ((0,), (0,)))

        @pl.when(i == 0)
        def _():
            sec_ref[...] = jnp.zeros_like(sec_ref)
            cs_ref[...] = jnp.zeros_like(cs_ref)

        sec_ref[...] += part
        cs_ref[...] += cs_part

        @pl.when(i == g - 1)
        def _():
            sec_ref[...] = sec_ref[...] / cs_ref[...]

    @pl.when((i >= g) & (i < 2 * g))
    def _phase_b():
        j = i - g
        x = x_ref[...]
        s = _dot(sec_ref[...], x, ((1,), (1,)))
        tile_max = jnp.max(s, axis=1, keepdims=True)

        @pl.when(j == 0)
        def _():
            m_ref[...] = jnp.full_like(m_ref, -jnp.inf)
            l_ref[...] = jnp.zeros_like(l_ref)
            acc_ref[...] = jnp.zeros_like(acc_ref)

        m_old = m_ref[...]
        m_new = jnp.maximum(m_old, tile_max)
        corr = jnp.exp(m_old - m_new)
        p = jnp.exp(s - m_new)
        l_ref[...] = l_ref[...] * corr + jnp.sum(p, axis=1, keepdims=True)
        acc_ref[...] = acc_ref[...] * corr + _dot(p, x, ((1,), (0,)))
        m_ref[...] = m_new

        @pl.when(j == g - 1)
        def _():
            sec2 = acc_ref[...] / l_ref[...]
            acc_ref[...] = sec2
            t1 = _dot(sec2, we_ref[...], ((1,), (1,)))
            mso_ref[...] = _dot(t1, wso_ref[...], ((1,), (1,)))
            t2 = _dot(t1, wsi_ref[...], ((1,), (1,)))
            mog_ref[...] = _dot(t2, wo_ref[...], ((1,), (1,)))
            vso_ref[...] = _dot(be_ref[...], wso_ref[...], ((1,), (1,))) + bso_ref[...]
            b1 = _dot(be_ref[...], wsi_ref[...], ((1,), (1,))) + bsi_ref[...]
            vg_ref[...] = _dot(b1, wo_ref[...], ((1,), (1,))) + bo_ref[...]

    @pl.when(i >= 2 * g)
    def _phase_c():
        x = x_ref[...]
        logits = _dot(x, acc_ref[...], ((1,), (1,)))
        logits = logits - jnp.max(logits, axis=1, keepdims=True)
        p = jnp.exp(logits)
        inv = p / jnp.sum(p, axis=1, keepdims=True)
        skip_out_ref[...] = _leaky(_dot(inv, mso_ref[...], ((1,), (0,))) + vso_ref[...])
        to_gnn_ref[...] = _leaky(_dot(x, wo_ref[...], ((1,), (1,)))
                                 + _dot(inv, mog_ref[...], ((1,), (0,))) + vg_ref[...])


@jax.jit
def kernel(x, ent2sec_mat, W_ent, b_ent, W_skip_in, b_skip_in,
           W_skip_out, b_skip_out, W_out, b_out):
    n, d = x.shape
    s = ent2sec_mat.shape[1]
    tile = _TILE if n % _TILE == 0 else n
    g = n // tile

    x_tile = lambda i: (jnp.where(i < g, i, jnp.where(i < 2 * g, i - g, i - 2 * g)), 0)
    e_tile = lambda i, kk: (jnp.where(i < g, 2 * i + kk, 2 * (g - 1) + kk), 0)
    out_tile = lambda i: (jnp.where(i >= 2 * g, i - 2 * g, 0), 0)
    whole = lambda i: (0, 0)

    bias2d = lambda b: b.reshape(1, d)
    wspec = pl.BlockSpec((d, d), whole)
    bspec = pl.BlockSpec((1, d), whole)
    sd = lambda: pltpu.VMEM((s, d), jnp.float32)
    s1 = lambda: pltpu.VMEM((s, 1), jnp.float32)
    v1 = lambda: pltpu.VMEM((1, d), jnp.float32)

    skip_out, to_gnn = pl.pallas_call(
        functools.partial(_fused_kernel, g=g, tile=tile),
        grid=(3 * g,),
        in_specs=[pl.BlockSpec((tile, d), x_tile)] + [
            pl.BlockSpec((tile // 2, s), functools.partial(e_tile, kk=k))
            for k in range(2)] + [
            wspec, bspec, wspec, bspec, wspec, bspec, wspec, bspec],
        out_specs=[pl.BlockSpec((tile, d), out_tile),
                   pl.BlockSpec((tile, d), out_tile)],
        out_shape=[jax.ShapeDtypeStruct((n, d), jnp.float32),
                   jax.ShapeDtypeStruct((n, d), jnp.float32)],
        scratch_shapes=[sd(), s1(), s1(), s1(), sd(), sd(), sd(), v1(), v1()],
    )(x, ent2sec_mat, ent2sec_mat,
      W_ent, bias2d(b_ent), W_skip_in, bias2d(b_skip_in),
      W_skip_out, bias2d(b_skip_out), W_out, bias2d(b_out))

    return (skip_out, to_gnn)
